# Initial kernel scaffold; baseline (speedup 1.0000x reference)
import jax
import jax.numpy as jnp
from jax.experimental import pallas as pl


def kernel(x, edge_index, conv1_weight, conv1_root, conv1_bias, conv2_weight, conv2_root, conv2_bias, fc1_w, fc1_b, fc2_w, fc2_b):
    raise NotImplementedError("write your pallas kernel here")


# speedup vs baseline: 10.1380x; 10.1380x over previous
import functools

import jax
import jax.numpy as jnp
from jax import lax
from jax.experimental import pallas as pl
from jax.experimental.pallas import tpu as pltpu
from jax.experimental.pallas import tpu_sc as plsc

NC, NS, LANES = 2, 16, 16
NW = NC * NS
EB = 128


def _make_seg_sum(R, D, NB, with_cnt):
  rpt = R // NS

  out_type = [jax.ShapeDtypeStruct((NC, R, D), jnp.float32)]
  scratch = [
      pltpu.VMEM_SHARED((R, D), jnp.float32),
      pltpu.VMEM((NB, EB), jnp.int32),
      pltpu.VMEM((NB, EB), jnp.int32),
      pltpu.VMEM((EB, D), jnp.float32),
  ]
  if with_cnt:
    out_type.append(jax.ShapeDtypeStruct((NC, R, LANES), jnp.float32))
    scratch += [
        pltpu.VMEM_SHARED((R, LANES), jnp.float32),
        pltpu.VMEM((EB, LANES), jnp.float32),
    ]

  mesh = plsc.VectorSubcoreMesh(core_axis_name="c", subcore_axis_name="s")

  @functools.partial(pl.kernel, out_type=tuple(out_type), mesh=mesh,
                     scratch_types=tuple(scratch),
                     compiler_params=pltpu.CompilerParams(
                         use_tc_tiling_on_sc=False))
  def seg(*refs):
    if with_cnt:
      (y_hbm, src_hbm, dst_hbm, zD_hbm, zc_hbm, ones_hbm,
       outS, outC, acc, src_v, dst_v, gbuf, cnt_acc, ones_v) = refs
    else:
      (y_hbm, src_hbm, dst_hbm, zD_hbm,
       outS, acc, src_v, dst_v, gbuf) = refs

    c = lax.axis_index("c")
    s = lax.axis_index("s")
    wid = s * NC + c
    r0 = s * rpt

    pltpu.sync_copy(zD_hbm.at[pl.ds(r0, rpt)], acc.at[pl.ds(r0, rpt)])
    if with_cnt:
      pltpu.sync_copy(zc_hbm.at[pl.ds(r0, rpt)], cnt_acc.at[pl.ds(r0, rpt)])
      pltpu.sync_copy(ones_hbm, ones_v)
    pltpu.sync_copy(src_hbm.at[wid], src_v)
    pltpu.sync_copy(dst_hbm.at[wid], dst_v)
    plsc.subcore_barrier()

    def ebody(j, carry):
      pltpu.sync_copy(y_hbm.at[src_v.at[j]], gbuf)
      pltpu.sync_copy(gbuf, acc.at[dst_v.at[j]], add=True)
      if with_cnt:
        pltpu.sync_copy(ones_v, cnt_acc.at[dst_v.at[j]], add=True)
      return carry
    lax.fori_loop(0, NB, ebody, 0)

    plsc.subcore_barrier()
    pltpu.sync_copy(acc.at[pl.ds(r0, rpt)], outS.at[c, pl.ds(r0, rpt)])
    if with_cnt:
      pltpu.sync_copy(cnt_acc.at[pl.ds(r0, rpt)], outC.at[c, pl.ds(r0, rpt)])

  return seg


def _mm_body(x_ref, w_ref, o_ref):
  o_ref[...] = jnp.dot(x_ref[...], w_ref[...],
                       preferred_element_type=jnp.float32)


def _stage_b_body(s1p_ref, cntp_ref, ycat_ref, b1_ref, w2_ref,
                  y2_ref, r2_ref):
  cnt = cntp_ref[0, :, 0:1] + cntp_ref[1, :, 0:1]
  inv = 1.0 / jnp.maximum(cnt, 1.0)
  s1 = s1p_ref[0] + s1p_ref[1]
  r1 = ycat_ref[:, 64:128]
  h = jnp.maximum(s1 * inv + r1 + b1_ref[...], 0.0)
  ycat2 = jnp.dot(h, w2_ref[...], preferred_element_type=jnp.float32)
  y2_ref[...] = ycat2[:, :32]
  r2_ref[...] = ycat2[:, 32:]


def _stage_c_body(s2p_ref, cntp_ref, r2_ref, b2_ref, fc1w_ref, fc1b_ref,
                  fc2w_ref, fc2b_ref, o_ref):
  cnt = cntp_ref[0, :, 0:1] + cntp_ref[1, :, 0:1]
  inv = 1.0 / jnp.maximum(cnt, 1.0)
  z = (s2p_ref[0] + s2p_ref[1]) * inv + r2_ref[...] + b2_ref[...]
  d = jnp.maximum(
      jnp.dot(z, fc1w_ref[...], preferred_element_type=jnp.float32)
      + fc1b_ref[...], 0.0)
  o_ref[...] = (jnp.dot(d, fc2w_ref[...], preferred_element_type=jnp.float32)
                + fc2b_ref[...])


def _row_spec(bm, width):
  return pl.BlockSpec((bm, width), lambda i: (i, 0))


def _full_spec(shape):
  nd = len(shape)
  return pl.BlockSpec(shape, lambda i: (0,) * nd)


def _part_spec(bm, width):
  return pl.BlockSpec((NC, bm, width), lambda i: (0, i, 0))


def kernel(x, edge_index, conv1_weight, conv1_root, conv1_bias,
           conv2_weight, conv2_root, conv2_bias, fc1_w, fc1_b, fc2_w, fc2_b):
  N, IN = x.shape
  E = edge_index.shape[1]
  H = conv1_root.shape[1]
  L = conv2_root.shape[1]

  R = ((N + NS - 1) // NS + 7) // 8 * 8 * NS
  NB = -(-E // (NW * EB))
  E_pad = NW * NB * EB

  src = jnp.concatenate(
      [edge_index[0], jnp.full((E_pad - E,), N, jnp.int32)]).reshape(NW, NB, EB)
  dst = jnp.concatenate(
      [edge_index[1], jnp.full((E_pad - E,), N, jnp.int32)]).reshape(NW, NB, EB)
  x_p = jnp.zeros((R, IN), x.dtype).at[:N].set(x)
  w1cat = jnp.concatenate([conv1_weight[0], conv1_root], axis=1)
  w2cat = jnp.concatenate([conv2_weight[0], conv2_root], axis=1)
  zH = jnp.zeros((R, H), jnp.float32)
  zL = jnp.zeros((R, L), jnp.float32)
  zc = jnp.zeros((R, LANES), jnp.float32)
  ones_rows = jnp.ones((EB, LANES), jnp.float32)

  bm = R // 4
  grid = (R // bm,)

  ycat1 = pl.pallas_call(
      _mm_body, grid=grid,
      in_specs=[_row_spec(bm, IN), _full_spec((IN, 2 * H))],
      out_specs=_row_spec(bm, 2 * H),
      out_shape=jax.ShapeDtypeStruct((R, 2 * H), jnp.float32),
  )(x_p, w1cat)

  y1 = ycat1[:, :H]
  seg1 = _make_seg_sum(R, H, NB, with_cnt=True)
  s1p, cntp = seg1(y1, src, dst, zH, zc, ones_rows)

  y2, r2 = pl.pallas_call(
      _stage_b_body, grid=grid,
      in_specs=[_part_spec(bm, H), _part_spec(bm, LANES), _row_spec(bm, 2 * H),
                _full_spec((1, H)), _full_spec((H, 2 * L))],
      out_specs=[_row_spec(bm, L), _row_spec(bm, L)],
      out_shape=[jax.ShapeDtypeStruct((R, L), jnp.float32),
                 jax.ShapeDtypeStruct((R, L), jnp.float32)],
  )(s1p, cntp, ycat1, conv1_bias.reshape(1, H), w2cat)

  seg2 = _make_seg_sum(R, L, NB, with_cnt=False)
  (s2p,) = seg2(y2, src, dst, zL)

  out = pl.pallas_call(
      _stage_c_body, grid=grid,
      in_specs=[_part_spec(bm, L), _part_spec(bm, LANES), _row_spec(bm, L),
                _full_spec((1, L)), _full_spec((L, H)), _full_spec((1, H)),
                _full_spec((H, IN)), _full_spec((1, IN))],
      out_specs=_row_spec(bm, IN),
      out_shape=jax.ShapeDtypeStruct((R, IN), jnp.float32),
  )(s2p, cntp, r2, conv2_bias.reshape(1, L), fc1_w, fc1_b.reshape(1, H),
    fc2_w, fc2_b.reshape(1, IN))

  return out[:N]
